# baseline (speedup 1.0000x reference)
import functools

import jax
import jax.numpy as jnp
from jax import lax
from jax.experimental import pallas as pl
from jax.experimental.pallas import tpu as pltpu
from jax.experimental.pallas import tpu_sc as plsc

_S = 30.0
_B = 16384
_C = 100
_L = 16
_NS = 16
_RW = _B // _NS
_G = _RW // _L
_BM = 2048
_NBT = _B // _BM


def _sc_body(xt_hbm, m_hbm, t_hbm, xt_out, bm_out, x_v, t_v, m_v, xo_v, bo_v):
    wid = lax.axis_index("s")
    base = wid * _RW
    pltpu.sync_copy(xt_hbm.at[:, pl.ds(base, _RW)], x_v)
    pltpu.sync_copy(t_hbm.at[pl.ds(base, _RW)], t_v)
    pltpu.sync_copy(m_hbm, m_v)
    lanes = lax.iota(jnp.int32, _L)

    def group(g, carry):
        r0 = g * _L
        rows = lanes + r0
        t = t_v[pl.ds(r0, _L)]
        bo_v[pl.ds(r0, _L)] = plsc.load_gather(m_v, [t])
        xo_v[pl.ds(r0, _L)] = plsc.load_gather(x_v, [t, rows])
        return carry

    lax.fori_loop(0, _G, group, 0)
    blk = wid // (_BM // _RW)
    off = (wid % (_BM // _RW)) * _RW
    pltpu.sync_copy(xo_v, xt_out.at[blk, 0, pl.ds(off, _RW)])
    pltpu.sync_copy(bo_v, bm_out.at[blk, 0, pl.ds(off, _RW)])


_sc_gather = functools.partial(
    pl.kernel,
    out_type=[
        jax.ShapeDtypeStruct((_NBT, 1, _BM), jnp.float32),
        jax.ShapeDtypeStruct((_NBT, 1, _BM), jnp.float32),
    ],
    mesh=plsc.VectorSubcoreMesh(
        core_axis_name="c", subcore_axis_name="s", num_cores=1, num_subcores=_NS
    ),
    scratch_types=[
        pltpu.VMEM((_C, _RW), jnp.float32),
        pltpu.VMEM((_RW,), jnp.int32),
        pltpu.VMEM((_C,), jnp.float32),
        pltpu.VMEM((_RW,), jnp.float32),
        pltpu.VMEM((_RW,), jnp.float32),
    ],
    compiler_params=pltpu.CompilerParams(needs_layout_passes=False),
)(_sc_body)


def _tc_block(xt_ref, xtg_ref, bm_ref, out_ref):
    i = pl.program_id(0)
    x = xt_ref[...]
    xt = xtg_ref[0]
    bm = bm_ref[0]
    mx = jnp.max(x, axis=0, keepdims=True)
    s = jnp.sum(jnp.exp((x - mx) * _S), axis=0, keepdims=True)
    e_raw = jnp.exp((xt - mx) * _S)
    e_mod = jnp.exp((xt - bm - mx) * _S)
    s2 = jnp.maximum(s - e_raw + e_mod, 1e-30)
    loss = jnp.log(s2) + _S * ((mx - xt) + bm)
    blk = jnp.sum(loss)

    @pl.when(i == 0)
    def _init():
        out_ref[0, 0] = 0.0

    out_ref[0, 0] += blk

    @pl.when(i == _NBT - 1)
    def _fin():
        out_ref[0, 0] = out_ref[0, 0] * (1.0 / _B)


def kernel(inputs, m_list, targets):
    xt = inputs.T
    xt_a, bm_a = _sc_gather(xt, m_list, targets)
    out = pl.pallas_call(
        _tc_block,
        grid=(_NBT,),
        in_specs=[
            pl.BlockSpec((_C, _BM), lambda i: (0, i)),
            pl.BlockSpec((1, 1, _BM), lambda i: (i, 0, 0)),
            pl.BlockSpec((1, 1, _BM), lambda i: (i, 0, 0)),
        ],
        out_specs=pl.BlockSpec((1, 1), lambda i: (0, 0), memory_space=pltpu.SMEM),
        out_shape=jax.ShapeDtypeStruct((1, 1), jnp.float32),
    )(xt, xt_a, bm_a)
    return out[0, 0]


# speedup vs baseline: 1.2035x; 1.2035x over previous
import functools

import jax
import jax.numpy as jnp
from jax import lax
from jax.experimental import pallas as pl
from jax.experimental.pallas import tpu as pltpu
from jax.experimental.pallas import tpu_sc as plsc

_S = 30.0
_B = 16384
_C = 100
_L = 16
_NS = 16
_RW = _B // _NS
_G = _RW // _L
_BM = 2048
_NBT = _B // _BM


def _sc_body(xt_hbm, m_hbm, t_hbm, xt_out, bm_out, x_v, t_v, m_v, xo_v, bo_v):
    wid = lax.axis_index("s")
    base = wid * _RW
    pltpu.sync_copy(xt_hbm.at[:, pl.ds(base, _RW)], x_v)
    pltpu.sync_copy(t_hbm.at[pl.ds(base, _RW)], t_v)
    pltpu.sync_copy(m_hbm, m_v)
    lanes = lax.iota(jnp.int32, _L)

    def group(g, carry):
        r0 = g * _L
        rows = lanes + r0
        t = t_v[pl.ds(r0, _L)]
        bo_v[pl.ds(r0, _L)] = plsc.load_gather(m_v, [t])
        xo_v[pl.ds(r0, _L)] = plsc.load_gather(x_v, [t, rows])
        return carry

    lax.fori_loop(0, _G, group, 0)
    blk = wid // (_BM // _RW)
    off = (wid % (_BM // _RW)) * _RW
    pltpu.sync_copy(xo_v, xt_out.at[blk, 0, pl.ds(off, _RW)])
    pltpu.sync_copy(bo_v, bm_out.at[blk, 0, pl.ds(off, _RW)])


_sc_gather = functools.partial(
    pl.kernel,
    out_type=[
        jax.ShapeDtypeStruct((_NBT, 1, _BM), jnp.float32),
        jax.ShapeDtypeStruct((_NBT, 1, _BM), jnp.float32),
    ],
    mesh=plsc.VectorSubcoreMesh(
        core_axis_name="c", subcore_axis_name="s", num_cores=1, num_subcores=_NS
    ),
    scratch_types=[
        pltpu.VMEM((_C, _RW), jnp.float32),
        pltpu.VMEM((_RW,), jnp.int32),
        pltpu.VMEM((_C,), jnp.float32),
        pltpu.VMEM((_RW,), jnp.float32),
        pltpu.VMEM((_RW,), jnp.float32),
    ],
    compiler_params=pltpu.CompilerParams(needs_layout_passes=False),
)(_sc_body)


def _tc_block(xt_ref, s_ref, mx_ref):
    x = xt_ref[...]
    mx = jnp.max(x, axis=0, keepdims=True)
    s = jnp.sum(jnp.exp((x - mx) * _S), axis=0, keepdims=True)
    s_ref[0] = s
    mx_ref[0] = mx


def _combine_body(s_ref, mx_ref, xt_ref, bm_ref, out_ref):
    s = s_ref[...]
    mx = mx_ref[...]
    xt = xt_ref[...]
    bm = bm_ref[...]
    e_raw = jnp.exp((xt - mx) * _S)
    e_mod = jnp.exp((xt - bm - mx) * _S)
    s2 = jnp.maximum(s - e_raw + e_mod, 1e-30)
    loss = jnp.log(s2) + _S * ((mx - xt) + bm)
    out_ref[0, 0] = jnp.sum(loss) * (1.0 / _B)


def kernel(inputs, m_list, targets):
    xt = inputs.T
    xt_a, bm_a = _sc_gather(xt, m_list, targets)
    s_a, mx_a = pl.pallas_call(
        _tc_block,
        grid=(_NBT,),
        in_specs=[pl.BlockSpec((_C, _BM), lambda i: (0, i))],
        out_specs=[
            pl.BlockSpec((1, 1, _BM), lambda i: (i, 0, 0)),
            pl.BlockSpec((1, 1, _BM), lambda i: (i, 0, 0)),
        ],
        out_shape=[
            jax.ShapeDtypeStruct((_NBT, 1, _BM), jnp.float32),
            jax.ShapeDtypeStruct((_NBT, 1, _BM), jnp.float32),
        ],
    )(xt)
    out = pl.pallas_call(
        _combine_body,
        in_specs=[
            pl.BlockSpec(memory_space=pltpu.VMEM),
            pl.BlockSpec(memory_space=pltpu.VMEM),
            pl.BlockSpec(memory_space=pltpu.VMEM),
            pl.BlockSpec(memory_space=pltpu.VMEM),
        ],
        out_specs=pl.BlockSpec(memory_space=pltpu.SMEM),
        out_shape=jax.ShapeDtypeStruct((1, 1), jnp.float32),
    )(s_a, mx_a, xt_a, bm_a)
    return out[0, 0]
